# baseline (speedup 1.0000x reference)
import functools

import jax
import jax.numpy as jnp
from jax.experimental import pallas as pl
from jax.experimental.pallas import tpu as pltpu

_VMEM_LIMIT_BYTES = 64 * 1024 * 1024


def _wce_body(x_ref, t_ref, w_ref, o_ref, lacc_ref, wacc_ref, *,
              n_classes, n_steps, block_n):
    j = pl.program_id(0)

    @pl.when(j == 0)
    def _():
        lacc_ref[...] = jnp.zeros_like(lacc_ref)
        wacc_ref[...] = jnp.zeros_like(wacc_ref)

    for b in range(block_n):
        t = t_ref[b, 0]

        m = x_ref[b, 0]
        for c in range(1, n_classes):
            m = jnp.maximum(m, x_ref[b, c])
        s = jnp.exp(x_ref[b, 0] - m)
        for c in range(1, n_classes):
            s = s + jnp.exp(x_ref[b, c] - m)
        lse = m + jnp.log(s)

        logit_t = jnp.zeros_like(m)
        w_t = jnp.zeros_like(m)
        for c in range(n_classes):
            hit = t == c
            logit_t = jnp.where(hit, x_ref[b, c], logit_t)
            w_t = jnp.where(hit, w_ref[c], w_t)

        lacc_ref[...] += w_t * (lse - logit_t)
        wacc_ref[...] += w_t

    @pl.when(j == n_steps - 1)
    def _():
        o_ref[0, 0] = jnp.sum(lacc_ref[...]) / jnp.sum(wacc_ref[...])


def kernel(inputs, targets, class_weights):
    n, c, h, w = inputs.shape
    t4 = targets.reshape(n, 1, h, w)
    cw = class_weights.astype(jnp.float32)

    block_n = 8
    n_steps = n // block_n

    out = pl.pallas_call(
        functools.partial(_wce_body, n_classes=c, n_steps=n_steps,
                          block_n=block_n),
        grid=(n_steps,),
        in_specs=[pl.BlockSpec((block_n, c, h, w), lambda j: (j, 0, 0, 0)),
                  pl.BlockSpec((block_n, 1, h, w), lambda j: (j, 0, 0, 0)),
                  pl.BlockSpec(memory_space=pltpu.MemorySpace.SMEM)],
        out_specs=pl.BlockSpec(memory_space=pltpu.MemorySpace.SMEM),
        out_shape=jax.ShapeDtypeStruct((1, 1), jnp.float32),
        scratch_shapes=[pltpu.VMEM((h, w), jnp.float32),
                        pltpu.VMEM((h, w), jnp.float32)],
        compiler_params=pltpu.CompilerParams(
            dimension_semantics=("arbitrary",),
            vmem_limit_bytes=_VMEM_LIMIT_BYTES),
    )(inputs, t4, cw)
    return out[0, 0]


# speedup vs baseline: 1.2269x; 1.2269x over previous
import functools

import jax
import jax.numpy as jnp
from jax.experimental import pallas as pl
from jax.experimental.pallas import tpu as pltpu

_VMEM_LIMIT_BYTES = 64 * 1024 * 1024
_N_BUFS = 4


def _wce_body(x_hbm, t_hbm, w_ref, o_ref,
              xbuf, tbuf, lacc_ref, wacc_ref, xsem, tsem, *,
              n_classes, n_images, n_bufs):
    def start_copy(img, slot):
        pltpu.make_async_copy(x_hbm.at[pl.ds(img, 1)],
                              xbuf.at[pl.ds(slot, 1)],
                              xsem.at[slot]).start()
        pltpu.make_async_copy(t_hbm.at[pl.ds(img, 1)],
                              tbuf.at[pl.ds(slot, 1)],
                              tsem.at[slot]).start()

    def wait_copy(slot):
        pltpu.make_async_copy(x_hbm.at[pl.ds(0, 1)],
                              xbuf.at[pl.ds(slot, 1)],
                              xsem.at[slot]).wait()
        pltpu.make_async_copy(t_hbm.at[pl.ds(0, 1)],
                              tbuf.at[pl.ds(slot, 1)],
                              tsem.at[slot]).wait()

    lacc_ref[...] = jnp.zeros_like(lacc_ref)
    wacc_ref[...] = jnp.zeros_like(wacc_ref)

    for i in range(min(n_bufs, n_images)):
        start_copy(i, i)

    def body(i, _):
        slot = jax.lax.rem(i, n_bufs)
        wait_copy(slot)
        x_s = xbuf.at[slot]
        t = tbuf[slot, 0]

        m = x_s[0]
        for c in range(1, n_classes):
            m = jnp.maximum(m, x_s[c])
        s = jnp.exp(x_s[0] - m)
        for c in range(1, n_classes):
            s = s + jnp.exp(x_s[c] - m)
        lse = m + jnp.log(s)

        logit_t = jnp.zeros_like(m)
        w_t = jnp.zeros_like(m)
        for c in range(n_classes):
            hit = t == c
            logit_t = jnp.where(hit, x_s[c], logit_t)
            w_t = jnp.where(hit, w_ref[c], w_t)

        lacc_ref[...] += w_t * (lse - logit_t)
        wacc_ref[...] += w_t

        @pl.when(i + n_bufs < n_images)
        def _():
            start_copy(i + n_bufs, slot)
        return ()

    jax.lax.fori_loop(0, n_images, body, ())
    o_ref[0, 0] = jnp.sum(lacc_ref[...]) / jnp.sum(wacc_ref[...])


def kernel(inputs, targets, class_weights):
    n, c, h, w = inputs.shape
    t4 = targets.reshape(n, 1, h, w)
    cw = class_weights.astype(jnp.float32)
    n_bufs = min(_N_BUFS, n)

    out = pl.pallas_call(
        functools.partial(_wce_body, n_classes=c, n_images=n, n_bufs=n_bufs),
        in_specs=[pl.BlockSpec(memory_space=pltpu.MemorySpace.HBM),
                  pl.BlockSpec(memory_space=pltpu.MemorySpace.HBM),
                  pl.BlockSpec(memory_space=pltpu.MemorySpace.SMEM)],
        out_specs=pl.BlockSpec(memory_space=pltpu.MemorySpace.SMEM),
        out_shape=jax.ShapeDtypeStruct((1, 1), jnp.float32),
        scratch_shapes=[pltpu.VMEM((n_bufs, c, h, w), jnp.float32),
                        pltpu.VMEM((n_bufs, 1, h, w), jnp.int32),
                        pltpu.VMEM((h, w), jnp.float32),
                        pltpu.VMEM((h, w), jnp.float32),
                        pltpu.SemaphoreType.DMA((n_bufs,)),
                        pltpu.SemaphoreType.DMA((n_bufs,))],
        compiler_params=pltpu.CompilerParams(
            vmem_limit_bytes=_VMEM_LIMIT_BYTES),
    )(inputs, t4, cw)
    return out[0, 0]
